# Initial kernel scaffold; baseline (speedup 1.0000x reference)
import jax
import jax.numpy as jnp
from jax.experimental import pallas as pl


def kernel(center_words, context_words, negative_words, W_in, W_out):
    raise NotImplementedError("write your pallas kernel here")


# speedup vs baseline: 4.0105x; 4.0105x over previous
import functools

import jax
import jax.numpy as jnp
from jax import lax
from jax.experimental import pallas as pl
from jax.experimental.pallas import tpu as pltpu
from jax.experimental.pallas import tpu_sc as plsc

VOCAB_N = 1000000
DIM = 64
BATCH = 16384
KNEG = 20

NCORE = 2
NSUB = 16
LANES = 16
NW = NCORE * NSUB
BPW = BATCH // NW
CHUNK = 32
NCHUNK = BPW // CHUNK
GROUPS = CHUNK // LANES
NEGC = CHUNK * KNEG
NIDX = 128
NSEG = NEGC // NIDX


def _skipgram_body(cen_hbm, ctx_hbm, neg_hbm, win_hbm, wout_hbm,
                   pos_out, neg_out,
                   cen_idx, ctx_idx, neg_idx,
                   cen_rows, ctx_rows, neg_rows,
                   pos_v, negsc_v, sem):
    cid = lax.axis_index("c")
    sid = lax.axis_index("s")
    wid = sid * NCORE + cid
    base = wid * BPW

    pltpu.sync_copy(cen_hbm.at[pl.ds(base, BPW)], cen_idx)
    pltpu.sync_copy(ctx_hbm.at[pl.ds(base, BPW)], ctx_idx)
    pltpu.sync_copy(neg_hbm.at[pl.ds(base * KNEG, BPW * KNEG)], neg_idx)

    iota = lax.iota(jnp.int32, LANES)

    def chunk_body(c, carry):
        cp1 = pltpu.async_copy(
            win_hbm.at[cen_idx.at[pl.ds(c * CHUNK, CHUNK)]], cen_rows, sem)
        cp2 = pltpu.async_copy(
            wout_hbm.at[ctx_idx.at[pl.ds(c * CHUNK, CHUNK)]], ctx_rows, sem)
        cps = [pltpu.async_copy(
                   wout_hbm.at[neg_idx.at[pl.ds(c * NEGC + q * NIDX, NIDX)]],
                   neg_rows.at[pl.ds(q * NIDX, NIDX), :], sem)
               for q in range(NSEG)]
        cp1.wait()
        cp2.wait()
        for cp in cps:
            cp.wait()

        for g in range(GROUPS):
            row = iota + (g * LANES)
            row20 = row * KNEG
            out_off = c * CHUNK + g * LANES

            def d_body(d, acc):
                pos, negs = acc
                dcol = jnp.full((LANES,), d, jnp.int32)
                cen_l = plsc.load_gather(cen_rows, [row, dcol])
                ctx_l = plsc.load_gather(ctx_rows, [row, dcol])
                pos = pos + cen_l * ctx_l
                new_negs = tuple(
                    negs[k] + plsc.load_gather(neg_rows, [row20 + k, dcol]) * cen_l
                    for k in range(KNEG))
                return (pos, new_negs)

            zero = jnp.zeros((LANES,), jnp.float32)
            pos, negs = lax.fori_loop(0, DIM, d_body, (zero, (zero,) * KNEG))

            pos_v[pl.ds(out_off, LANES)] = pos
            orow = iota + out_off
            for k in range(KNEG):
                plsc.store_scatter(
                    negsc_v, [orow, jnp.full((LANES,), k, jnp.int32)], negs[k])
        return carry

    lax.fori_loop(0, NCHUNK, chunk_body, 0)

    pltpu.sync_copy(pos_v, pos_out.at[pl.ds(base, BPW)])
    pltpu.sync_copy(negsc_v, neg_out.at[pl.ds(base, BPW)])


@functools.partial(
    pl.kernel,
    mesh=plsc.VectorSubcoreMesh(core_axis_name="c", subcore_axis_name="s"),
    out_type=(jax.ShapeDtypeStruct((BATCH,), jnp.float32),
              jax.ShapeDtypeStruct((BATCH, KNEG), jnp.float32)),
    scratch_types=[
        pltpu.VMEM((BPW,), jnp.int32),
        pltpu.VMEM((BPW,), jnp.int32),
        pltpu.VMEM((BPW * KNEG,), jnp.int32),
        pltpu.VMEM((CHUNK, DIM), jnp.float32),
        pltpu.VMEM((CHUNK, DIM), jnp.float32),
        pltpu.VMEM((NEGC, DIM), jnp.float32),
        pltpu.VMEM((BPW,), jnp.float32),
        pltpu.VMEM((BPW, KNEG), jnp.float32),
        pltpu.SemaphoreType.DMA,
    ],
    compiler_params=pltpu.CompilerParams(
        use_tc_tiling_on_sc=False, needs_layout_passes=False),
)
def _skipgram_sc(cen_hbm, ctx_hbm, neg_hbm, win_hbm, wout_hbm,
                 pos_out, neg_out, *rest):
    _skipgram_body(cen_hbm, ctx_hbm, neg_hbm, win_hbm, wout_hbm,
                   pos_out, neg_out, *rest)


def kernel(center_words, context_words, negative_words, W_in, W_out):
    cen = center_words.astype(jnp.int32)
    ctx = context_words.astype(jnp.int32)
    neg = negative_words.astype(jnp.int32).reshape(BATCH * KNEG)
    pos, negsc = _skipgram_sc(cen, ctx, neg, W_in, W_out)
    return (pos, negsc)


# speedup vs baseline: 5.3118x; 1.3245x over previous
import functools

import jax
import jax.numpy as jnp
from jax import lax
from jax.experimental import pallas as pl
from jax.experimental.pallas import tpu as pltpu
from jax.experimental.pallas import tpu_sc as plsc

VOCAB_N = 1000000
DIM = 64
BATCH = 16384
KNEG = 20

NCORE = 2
NSUB = 16
LANES = 16
DREG = DIM // LANES
NW = NCORE * NSUB
BPW = BATCH // NW
CHUNK = 32
NCHUNK = BPW // CHUNK
NEGC = CHUNK * KNEG
NIDX = 128
NSEG = NEGC // NIDX


def _skipgram_body(cen_hbm, ctx_hbm, neg_hbm, win_hbm, wout_hbm,
                   pos_out, neg_out,
                   cen_idx, ctx_idx, neg_idx,
                   cen0, ctx0, neg0, cen1, ctx1, neg1,
                   pos_v, negsc_v, sem0, sem1):
    cid = lax.axis_index("c")
    sid = lax.axis_index("s")
    wid = sid * NCORE + cid
    base = wid * BPW

    pltpu.sync_copy(cen_hbm.at[pl.ds(base, BPW)], cen_idx)
    pltpu.sync_copy(ctx_hbm.at[pl.ds(base, BPW)], ctx_idx)
    pltpu.sync_copy(neg_hbm.at[pl.ds(base * KNEG, BPW * KNEG)], neg_idx)

    iota = lax.iota(jnp.int32, LANES)
    m15 = iota == (LANES - 1)

    slots = ((cen0, ctx0, neg0, sem0), (cen1, ctx1, neg1, sem1))

    def issue(c, slot):
        cen_s, ctx_s, neg_s, sem = slots[slot]
        pltpu.async_copy(
            win_hbm.at[cen_idx.at[pl.ds(c * CHUNK, CHUNK)]], cen_s, sem)
        pltpu.async_copy(
            wout_hbm.at[ctx_idx.at[pl.ds(c * CHUNK, CHUNK)]], ctx_s, sem)
        for q in range(NSEG):
            pltpu.async_copy(
                wout_hbm.at[neg_idx.at[pl.ds(c * NEGC + q * NIDX, NIDX)]],
                neg_s.at[pl.ds(q * NIDX, NIDX), :], sem)

    def drain(slot):
        cen_s, ctx_s, neg_s, sem = slots[slot]
        pltpu.make_async_copy(win_hbm.at[pl.ds(0, CHUNK)], cen_s, sem).wait()
        pltpu.make_async_copy(wout_hbm.at[pl.ds(0, CHUNK)], ctx_s, sem).wait()
        pltpu.make_async_copy(wout_hbm.at[pl.ds(0, NEGC)], neg_s, sem).wait()

    def compute(c, slot):
        cen_s, ctx_s, neg_s, _ = slots[slot]

        @plsc.parallel_loop(0, CHUNK, 1, unroll=2)
        def bi_body(bi):
            off = c * CHUNK + bi
            offv = jnp.full((LANES,), off, jnp.int32)
            cvecs = [cen_s[bi, pl.ds(t * LANES, LANES)] for t in range(DREG)]
            pv = ctx_s[bi, pl.ds(0, LANES)] * cvecs[0]
            for t in range(1, DREG):
                pv = pv + ctx_s[bi, pl.ds(t * LANES, LANES)] * cvecs[t]
            plsc.store_scatter(pos_v, [offv], plsc.cumsum(pv), mask=m15)
            r0 = bi * KNEG
            for k in range(KNEG):
                nv = neg_s[r0 + k, pl.ds(0, LANES)] * cvecs[0]
                for t in range(1, DREG):
                    nv = nv + neg_s[r0 + k, pl.ds(t * LANES, LANES)] * cvecs[t]
                plsc.store_scatter(
                    negsc_v, [offv, jnp.full((LANES,), k, jnp.int32)],
                    plsc.cumsum(nv), mask=m15)

    issue(0, 0)

    def super_body(s, carry):
        c0 = s * 2
        issue(c0 + 1, 1)
        drain(0)
        compute(c0, 0)

        @pl.when(c0 + 2 < NCHUNK)
        def _():
            issue(c0 + 2, 0)

        drain(1)
        compute(c0 + 1, 1)
        return carry

    lax.fori_loop(0, NCHUNK // 2, super_body, 0)

    pltpu.sync_copy(pos_v, pos_out.at[pl.ds(base, BPW)])
    pltpu.sync_copy(negsc_v, neg_out.at[pl.ds(base, BPW)])


@functools.partial(
    pl.kernel,
    mesh=plsc.VectorSubcoreMesh(core_axis_name="c", subcore_axis_name="s"),
    out_type=(jax.ShapeDtypeStruct((BATCH,), jnp.float32),
              jax.ShapeDtypeStruct((BATCH, KNEG), jnp.float32)),
    scratch_types=[
        pltpu.VMEM((BPW,), jnp.int32),
        pltpu.VMEM((BPW,), jnp.int32),
        pltpu.VMEM((BPW * KNEG,), jnp.int32),
        pltpu.VMEM((CHUNK, DIM), jnp.float32),
        pltpu.VMEM((CHUNK, DIM), jnp.float32),
        pltpu.VMEM((NEGC, DIM), jnp.float32),
        pltpu.VMEM((CHUNK, DIM), jnp.float32),
        pltpu.VMEM((CHUNK, DIM), jnp.float32),
        pltpu.VMEM((NEGC, DIM), jnp.float32),
        pltpu.VMEM((BPW,), jnp.float32),
        pltpu.VMEM((BPW, KNEG), jnp.float32),
        pltpu.SemaphoreType.DMA,
        pltpu.SemaphoreType.DMA,
    ],
    compiler_params=pltpu.CompilerParams(
        use_tc_tiling_on_sc=False, needs_layout_passes=False),
)
def _skipgram_sc(cen_hbm, ctx_hbm, neg_hbm, win_hbm, wout_hbm,
                 pos_out, neg_out, *rest):
    _skipgram_body(cen_hbm, ctx_hbm, neg_hbm, win_hbm, wout_hbm,
                   pos_out, neg_out, *rest)


def kernel(center_words, context_words, negative_words, W_in, W_out):
    cen = center_words.astype(jnp.int32)
    ctx = context_words.astype(jnp.int32)
    neg = negative_words.astype(jnp.int32).reshape(BATCH * KNEG)
    pos, negsc = _skipgram_sc(cen, ctx, neg, W_in, W_out)
    return (pos, negsc)


# speedup vs baseline: 6.7116x; 1.2635x over previous
import functools

import jax
import jax.numpy as jnp
from jax import lax
from jax.experimental import pallas as pl
from jax.experimental.pallas import tpu as pltpu
from jax.experimental.pallas import tpu_sc as plsc

VOCAB_N = 1000000
DIM = 64
BATCH = 16384
KNEG = 20

NCORE = 2
NSUB = 16
LANES = 16
DREG = DIM // LANES
NW = NCORE * NSUB
BPW = BATCH // NW
CHUNK = 32
NCHUNK = BPW // CHUNK
NEGC = CHUNK * KNEG
NIDX = 128
NSEG = NEGC // NIDX


def _skipgram_body(cen_emb_hbm, ctx_hbm, neg_hbm, wout_hbm,
                   pos_out, neg_out,
                   ctx_idx, neg_idx,
                   cen0, ctx0, neg0, cen1, ctx1, neg1,
                   pos_v, negsc_v, sem0, sem1):
    cid = lax.axis_index("c")
    sid = lax.axis_index("s")
    wid = sid * NCORE + cid
    base = wid * BPW

    pltpu.sync_copy(ctx_hbm.at[pl.ds(base, BPW)], ctx_idx)
    pltpu.sync_copy(neg_hbm.at[pl.ds(base * KNEG, BPW * KNEG)], neg_idx)

    iota = lax.iota(jnp.int32, LANES)
    m15 = iota == (LANES - 1)

    slots = ((cen0, ctx0, neg0, sem0), (cen1, ctx1, neg1, sem1))

    def issue(c, slot):
        cen_s, ctx_s, neg_s, sem = slots[slot]
        pltpu.async_copy(
            cen_emb_hbm.at[pl.ds(base + c * CHUNK, CHUNK)], cen_s, sem)
        pltpu.async_copy(
            wout_hbm.at[ctx_idx.at[pl.ds(c * CHUNK, CHUNK)]], ctx_s, sem)
        for q in range(NSEG):
            pltpu.async_copy(
                wout_hbm.at[neg_idx.at[pl.ds(c * NEGC + q * NIDX, NIDX)]],
                neg_s.at[pl.ds(q * NIDX, NIDX), :], sem)

    def drain(slot):
        cen_s, ctx_s, neg_s, sem = slots[slot]
        pltpu.make_async_copy(
            cen_emb_hbm.at[pl.ds(0, CHUNK)], cen_s, sem).wait()
        pltpu.make_async_copy(wout_hbm.at[pl.ds(0, CHUNK)], ctx_s, sem).wait()
        pltpu.make_async_copy(wout_hbm.at[pl.ds(0, NEGC)], neg_s, sem).wait()

    def compute(c, slot):
        cen_s, ctx_s, neg_s, _ = slots[slot]

        @plsc.parallel_loop(0, CHUNK, 1, unroll=2)
        def bi_body(bi):
            off = c * CHUNK + bi
            offv = jnp.full((LANES,), off, jnp.int32)
            cvecs = [cen_s[bi, pl.ds(t * LANES, LANES)] for t in range(DREG)]
            pv = ctx_s[bi, pl.ds(0, LANES)] * cvecs[0]
            for t in range(1, DREG):
                pv = pv + ctx_s[bi, pl.ds(t * LANES, LANES)] * cvecs[t]
            plsc.store_scatter(pos_v, [offv], plsc.cumsum(pv), mask=m15)
            r0 = bi * KNEG
            for k in range(KNEG):
                nv = neg_s[r0 + k, pl.ds(0, LANES)] * cvecs[0]
                for t in range(1, DREG):
                    nv = nv + neg_s[r0 + k, pl.ds(t * LANES, LANES)] * cvecs[t]
                plsc.store_scatter(
                    negsc_v, [offv, jnp.full((LANES,), k, jnp.int32)],
                    plsc.cumsum(nv), mask=m15)

        del bi_body

    issue(0, 0)

    def super_body(s, carry):
        c0 = s * 2
        issue(c0 + 1, 1)
        drain(0)
        compute(c0, 0)

        @pl.when(c0 + 2 < NCHUNK)
        def _():
            issue(c0 + 2, 0)

        drain(1)
        compute(c0 + 1, 1)
        return carry

    lax.fori_loop(0, NCHUNK // 2, super_body, 0)

    pltpu.sync_copy(pos_v, pos_out.at[pl.ds(base, BPW)])
    pltpu.sync_copy(negsc_v, neg_out.at[pl.ds(base, BPW)])


@functools.partial(
    pl.kernel,
    mesh=plsc.VectorSubcoreMesh(core_axis_name="c", subcore_axis_name="s"),
    out_type=(jax.ShapeDtypeStruct((BATCH,), jnp.float32),
              jax.ShapeDtypeStruct((BATCH, KNEG), jnp.float32)),
    scratch_types=[
        pltpu.VMEM((BPW,), jnp.int32),
        pltpu.VMEM((BPW * KNEG,), jnp.int32),
        pltpu.VMEM((CHUNK, DIM), jnp.float32),
        pltpu.VMEM((CHUNK, DIM), jnp.float32),
        pltpu.VMEM((NEGC, DIM), jnp.float32),
        pltpu.VMEM((CHUNK, DIM), jnp.float32),
        pltpu.VMEM((CHUNK, DIM), jnp.float32),
        pltpu.VMEM((NEGC, DIM), jnp.float32),
        pltpu.VMEM((BPW,), jnp.float32),
        pltpu.VMEM((BPW, KNEG), jnp.float32),
        pltpu.SemaphoreType.DMA,
        pltpu.SemaphoreType.DMA,
    ],
    compiler_params=pltpu.CompilerParams(
        use_tc_tiling_on_sc=False, needs_layout_passes=False),
)
def _skipgram_sc(cen_emb_hbm, ctx_hbm, neg_hbm, wout_hbm,
                 pos_out, neg_out, *rest):
    _skipgram_body(cen_emb_hbm, ctx_hbm, neg_hbm, wout_hbm,
                   pos_out, neg_out, *rest)


def kernel(center_words, context_words, negative_words, W_in, W_out):
    ctx = context_words.astype(jnp.int32)
    neg = negative_words.astype(jnp.int32).reshape(BATCH * KNEG)
    cen_emb = jnp.take(W_in, center_words, axis=0)
    pos, negsc = _skipgram_sc(cen_emb, ctx, neg, W_out)
    return (pos, negsc)


# speedup vs baseline: 7.9505x; 1.1846x over previous
import functools

import jax
import jax.numpy as jnp
from jax import lax
from jax.experimental import pallas as pl
from jax.experimental.pallas import tpu as pltpu
from jax.experimental.pallas import tpu_sc as plsc

VOCAB_N = 1000000
DIM = 64
BATCH = 16384
KNEG = 20

NCORE = 2
NSUB = 16
LANES = 16
DREG = DIM // LANES
NW = NCORE * NSUB
BPW = BATCH // NW
CHUNK = 16
NCHUNK = BPW // CHUNK
NEGC = CHUNK * KNEG


def _skipgram_body(cen_emb_hbm, ctx_hbm, neg_hbm, wout_hbm,
                   pos_out, neg_out,
                   ctx_idx, neg_idx,
                   cen0, ctx0, neg0, cen1, ctx1, neg1,
                   pos_v, negsc_v, sem0, sem1):
    cid = lax.axis_index("c")
    sid = lax.axis_index("s")
    wid = sid * NCORE + cid
    base = wid * BPW

    pltpu.sync_copy(ctx_hbm.at[pl.ds(base, BPW)], ctx_idx)
    pltpu.sync_copy(neg_hbm.at[pl.ds(base * KNEG, BPW * KNEG)], neg_idx)

    iota = lax.iota(jnp.int32, LANES)
    m15 = iota == (LANES - 1)

    slots = ((cen0, ctx0, neg0, sem0), (cen1, ctx1, neg1, sem1))

    def issue(c, slot):
        cen_s, ctx_s, neg_s, sem = slots[slot]
        pltpu.async_copy(
            cen_emb_hbm.at[pl.ds(base + c * CHUNK, CHUNK)], cen_s, sem)

        vidx = ctx_idx[pl.ds(c * CHUNK, LANES)]
        for j in range(LANES):
            pltpu.async_copy(
                wout_hbm.at[pl.ds(vidx[j], 1)], ctx_s.at[pl.ds(j, 1)], sem)

        def neg_issue(g, carry):
            vidx = neg_idx[pl.ds(c * NEGC + g * LANES, LANES)]
            for j in range(LANES):
                pltpu.async_copy(
                    wout_hbm.at[pl.ds(vidx[j], 1)],
                    neg_s.at[pl.ds(g * LANES + j, 1)], sem)
            return carry

        lax.fori_loop(0, NEGC // LANES, neg_issue, 0)

    def drain(slot):
        cen_s, ctx_s, neg_s, sem = slots[slot]
        pltpu.make_async_copy(
            cen_emb_hbm.at[pl.ds(0, CHUNK)], cen_s, sem).wait()

        def ctx_drain(j, carry):
            pltpu.make_async_copy(
                wout_hbm.at[pl.ds(0, 1)], ctx_s.at[pl.ds(j, 1)], sem).wait()
            return carry

        lax.fori_loop(0, CHUNK, ctx_drain, 0)

        def neg_drain(j, carry):
            pltpu.make_async_copy(
                wout_hbm.at[pl.ds(0, 1)], neg_s.at[pl.ds(j, 1)], sem).wait()
            return carry

        lax.fori_loop(0, NEGC, neg_drain, 0)

    def compute(c, slot):
        cen_s, ctx_s, neg_s, _ = slots[slot]

        @plsc.parallel_loop(0, CHUNK, 1, unroll=2)
        def bi_body(bi):
            off = c * CHUNK + bi
            offv = jnp.full((LANES,), off, jnp.int32)
            cvecs = [cen_s[bi, pl.ds(t * LANES, LANES)] for t in range(DREG)]
            pv = ctx_s[bi, pl.ds(0, LANES)] * cvecs[0]
            for t in range(1, DREG):
                pv = pv + ctx_s[bi, pl.ds(t * LANES, LANES)] * cvecs[t]
            plsc.store_scatter(pos_v, [offv], plsc.cumsum(pv), mask=m15)
            r0 = bi * KNEG
            for k in range(KNEG):
                nv = neg_s[r0 + k, pl.ds(0, LANES)] * cvecs[0]
                for t in range(1, DREG):
                    nv = nv + neg_s[r0 + k, pl.ds(t * LANES, LANES)] * cvecs[t]
                plsc.store_scatter(
                    negsc_v, [offv * KNEG + k], plsc.cumsum(nv), mask=m15)

        del bi_body

    issue(0, 0)

    def super_body(s, carry):
        c0 = s * 2
        issue(c0 + 1, 1)
        drain(0)
        compute(c0, 0)

        @pl.when(c0 + 2 < NCHUNK)
        def _():
            issue(c0 + 2, 0)

        drain(1)
        compute(c0 + 1, 1)
        return carry

    lax.fori_loop(0, NCHUNK // 2, super_body, 0)

    pltpu.sync_copy(pos_v, pos_out.at[pl.ds(base, BPW)])
    pltpu.sync_copy(negsc_v, neg_out.at[pl.ds(base * KNEG, BPW * KNEG)])


@functools.partial(
    pl.kernel,
    mesh=plsc.VectorSubcoreMesh(core_axis_name="c", subcore_axis_name="s"),
    out_type=(jax.ShapeDtypeStruct((BATCH,), jnp.float32),
              jax.ShapeDtypeStruct((BATCH * KNEG,), jnp.float32)),
    scratch_types=[
        pltpu.VMEM((BPW,), jnp.int32),
        pltpu.VMEM((BPW * KNEG,), jnp.int32),
        pltpu.VMEM((CHUNK, DIM), jnp.float32),
        pltpu.VMEM((CHUNK, DIM), jnp.float32),
        pltpu.VMEM((NEGC, DIM), jnp.float32),
        pltpu.VMEM((CHUNK, DIM), jnp.float32),
        pltpu.VMEM((CHUNK, DIM), jnp.float32),
        pltpu.VMEM((NEGC, DIM), jnp.float32),
        pltpu.VMEM((BPW,), jnp.float32),
        pltpu.VMEM((BPW * KNEG,), jnp.float32),
        pltpu.SemaphoreType.DMA,
        pltpu.SemaphoreType.DMA,
    ],
    compiler_params=pltpu.CompilerParams(
        use_tc_tiling_on_sc=True, needs_layout_passes=False),
)
def _skipgram_sc(cen_emb_hbm, ctx_hbm, neg_hbm, wout_hbm,
                 pos_out, neg_out, *rest):
    _skipgram_body(cen_emb_hbm, ctx_hbm, neg_hbm, wout_hbm,
                   pos_out, neg_out, *rest)


def kernel(center_words, context_words, negative_words, W_in, W_out):
    ctx = context_words.astype(jnp.int32)
    neg = negative_words.astype(jnp.int32).reshape(BATCH * KNEG)
    cen_emb = jnp.take(W_in, center_words, axis=0)
    pos, negsc = _skipgram_sc(cen_emb, ctx, neg, W_out)
    return (pos, negsc.reshape(BATCH, KNEG))


# speedup vs baseline: 8.3682x; 1.0525x over previous
import functools

import jax
import jax.numpy as jnp
from jax import lax
from jax.experimental import pallas as pl
from jax.experimental.pallas import tpu as pltpu
from jax.experimental.pallas import tpu_sc as plsc

VOCAB_N = 1000000
DIM = 64
BATCH = 16384
KNEG = 20

NCORE = 2
NSUB = 16
LANES = 16
DREG = DIM // LANES
NW = NCORE * NSUB
BPW = BATCH // NW
CHUNK = 16
NCHUNK = BPW // CHUNK
NEGC = CHUNK * KNEG


def _skipgram_body(cen_emb_hbm, ctx_hbm, neg_hbm, wout_hbm,
                   pos_out, neg_out,
                   ctx_idx, neg_idx,
                   cen0, ctx0, neg0, cen1, ctx1, neg1,
                   pos_v, negsc_v, sem0, sem1):
    cid = lax.axis_index("c")
    sid = lax.axis_index("s")
    wid = sid * NCORE + cid
    base = wid * BPW

    pltpu.sync_copy(ctx_hbm.at[pl.ds(base, BPW)], ctx_idx)
    pltpu.sync_copy(neg_hbm.at[pl.ds(base * KNEG, BPW * KNEG)], neg_idx)

    iota = lax.iota(jnp.int32, LANES)
    m15 = iota == (LANES - 1)

    slots = ((cen0, ctx0, neg0, sem0), (cen1, ctx1, neg1, sem1))

    def issue(c, slot):
        cen_s, ctx_s, neg_s, sem = slots[slot]
        pltpu.async_copy(
            cen_emb_hbm.at[pl.ds(base + c * CHUNK, CHUNK)], cen_s, sem)

        vidx = ctx_idx[pl.ds(c * CHUNK, LANES)]
        for j in range(LANES):
            pltpu.async_copy(
                wout_hbm.at[pl.ds(vidx[j], 1)], ctx_s.at[pl.ds(j, 1)], sem)

        def neg_issue(g, carry):
            vidx = neg_idx[pl.ds(c * NEGC + g * LANES, LANES)]
            for j in range(LANES):
                pltpu.async_copy(
                    wout_hbm.at[pl.ds(vidx[j], 1)],
                    neg_s.at[pl.ds(g * LANES + j, 1)], sem)
            return carry

        lax.fori_loop(0, NEGC // LANES, neg_issue, 0)

    def drain(slot):
        cen_s, ctx_s, neg_s, sem = slots[slot]
        pltpu.make_async_copy(
            cen_emb_hbm.at[pl.ds(0, CHUNK)], cen_s, sem).wait()
        pltpu.make_async_copy(wout_hbm.at[pl.ds(0, CHUNK)], ctx_s, sem).wait()
        pltpu.make_async_copy(wout_hbm.at[pl.ds(0, NEGC)], neg_s, sem).wait()

    def compute(c, slot):
        cen_s, ctx_s, neg_s, _ = slots[slot]

        @plsc.parallel_loop(0, CHUNK, 1, unroll=2)
        def bi_body(bi):
            off = c * CHUNK + bi
            offv = jnp.full((LANES,), off, jnp.int32)
            cvecs = [cen_s[bi, pl.ds(t * LANES, LANES)] for t in range(DREG)]
            pv = ctx_s[bi, pl.ds(0, LANES)] * cvecs[0]
            for t in range(1, DREG):
                pv = pv + ctx_s[bi, pl.ds(t * LANES, LANES)] * cvecs[t]
            plsc.store_scatter(pos_v, [offv], plsc.cumsum(pv), mask=m15)
            r0 = bi * KNEG
            for k in range(KNEG):
                nv = neg_s[r0 + k, pl.ds(0, LANES)] * cvecs[0]
                for t in range(1, DREG):
                    nv = nv + neg_s[r0 + k, pl.ds(t * LANES, LANES)] * cvecs[t]
                plsc.store_scatter(
                    negsc_v, [offv * KNEG + k], plsc.cumsum(nv), mask=m15)

        del bi_body

    issue(0, 0)

    def super_body(s, carry):
        c0 = s * 2
        issue(c0 + 1, 1)
        drain(0)
        compute(c0, 0)

        @pl.when(c0 + 2 < NCHUNK)
        def _():
            issue(c0 + 2, 0)

        drain(1)
        compute(c0 + 1, 1)
        return carry

    lax.fori_loop(0, NCHUNK // 2, super_body, 0)

    pltpu.sync_copy(pos_v, pos_out.at[pl.ds(base, BPW)])
    pltpu.sync_copy(negsc_v, neg_out.at[pl.ds(base * KNEG, BPW * KNEG)])


@functools.partial(
    pl.kernel,
    mesh=plsc.VectorSubcoreMesh(core_axis_name="c", subcore_axis_name="s"),
    out_type=(jax.ShapeDtypeStruct((BATCH,), jnp.float32),
              jax.ShapeDtypeStruct((BATCH * KNEG,), jnp.float32)),
    scratch_types=[
        pltpu.VMEM((BPW,), jnp.int32),
        pltpu.VMEM((BPW * KNEG,), jnp.int32),
        pltpu.VMEM((CHUNK, DIM), jnp.float32),
        pltpu.VMEM((CHUNK, DIM), jnp.float32),
        pltpu.VMEM((NEGC, DIM), jnp.float32),
        pltpu.VMEM((CHUNK, DIM), jnp.float32),
        pltpu.VMEM((CHUNK, DIM), jnp.float32),
        pltpu.VMEM((NEGC, DIM), jnp.float32),
        pltpu.VMEM((BPW,), jnp.float32),
        pltpu.VMEM((BPW * KNEG,), jnp.float32),
        pltpu.SemaphoreType.DMA,
        pltpu.SemaphoreType.DMA,
    ],
    compiler_params=pltpu.CompilerParams(
        use_tc_tiling_on_sc=True, needs_layout_passes=False),
)
def _skipgram_sc(cen_emb_hbm, ctx_hbm, neg_hbm, wout_hbm,
                 pos_out, neg_out, *rest):
    _skipgram_body(cen_emb_hbm, ctx_hbm, neg_hbm, wout_hbm,
                   pos_out, neg_out, *rest)


def kernel(center_words, context_words, negative_words, W_in, W_out):
    ctx = context_words.astype(jnp.int32)
    neg = negative_words.astype(jnp.int32).reshape(BATCH * KNEG)
    cen_emb = jnp.take(W_in, center_words, axis=0)
    pos, negsc = _skipgram_sc(cen_emb, ctx, neg, W_out)
    return (pos, negsc.reshape(BATCH, KNEG))


# speedup vs baseline: 9.3774x; 1.1206x over previous
import functools

import jax
import jax.numpy as jnp
from jax import lax
from jax.experimental import pallas as pl
from jax.experimental.pallas import tpu as pltpu
from jax.experimental.pallas import tpu_sc as plsc

VOCAB_N = 1000000
DIM = 64
BATCH = 16384
KNEG = 20

NCORE = 2
NSUB = 16
LANES = 16
DREG = DIM // LANES
NW = NCORE * NSUB
BPW = BATCH // NW
CHUNK = 16
NCHUNK = BPW // CHUNK
NEGC = CHUNK * KNEG


def _skipgram_body(cen_emb_hbm, ctx_emb_hbm, neg_hbm, wout_hbm,
                   pos_out, neg_out,
                   neg_idx,
                   cen0, ctx0, neg0, cen1, ctx1, neg1,
                   pos_v, negsc_v, sem0, sem1):
    cid = lax.axis_index("c")
    sid = lax.axis_index("s")
    wid = sid * NCORE + cid
    base = wid * BPW

    pltpu.sync_copy(neg_hbm.at[pl.ds(base * KNEG, BPW * KNEG)], neg_idx)

    iota = lax.iota(jnp.int32, LANES)
    m15 = iota == (LANES - 1)

    slots = ((cen0, ctx0, neg0, sem0), (cen1, ctx1, neg1, sem1))

    def issue(c, slot):
        cen_s, ctx_s, neg_s, sem = slots[slot]
        pltpu.async_copy(
            cen_emb_hbm.at[pl.ds(base + c * CHUNK, CHUNK)], cen_s, sem)
        pltpu.async_copy(
            ctx_emb_hbm.at[pl.ds(base + c * CHUNK, CHUNK)], ctx_s, sem)

        def neg_issue(g, carry):
            vidx = neg_idx[pl.ds(c * NEGC + g * LANES, LANES)]
            for j in range(LANES):
                pltpu.async_copy(
                    wout_hbm.at[pl.ds(vidx[j], 1)],
                    neg_s.at[pl.ds(g * LANES + j, 1)], sem)
            return carry

        lax.fori_loop(0, NEGC // LANES, neg_issue, 0)

    def drain(slot):
        cen_s, ctx_s, neg_s, sem = slots[slot]
        pltpu.make_async_copy(
            cen_emb_hbm.at[pl.ds(0, CHUNK)], cen_s, sem).wait()
        pltpu.make_async_copy(
            ctx_emb_hbm.at[pl.ds(0, CHUNK)], ctx_s, sem).wait()
        pltpu.make_async_copy(wout_hbm.at[pl.ds(0, NEGC)], neg_s, sem).wait()

    def compute(c, slot):
        cen_s, ctx_s, neg_s, _ = slots[slot]

        @plsc.parallel_loop(0, CHUNK, 1, unroll=2)
        def bi_body(bi):
            off = c * CHUNK + bi
            offv = jnp.full((LANES,), off, jnp.int32)
            cvecs = [cen_s[bi, pl.ds(t * LANES, LANES)] for t in range(DREG)]
            pv = ctx_s[bi, pl.ds(0, LANES)] * cvecs[0]
            for t in range(1, DREG):
                pv = pv + ctx_s[bi, pl.ds(t * LANES, LANES)] * cvecs[t]
            plsc.store_scatter(pos_v, [offv], plsc.cumsum(pv), mask=m15)
            r0 = bi * KNEG
            for k in range(KNEG):
                nv = neg_s[r0 + k, pl.ds(0, LANES)] * cvecs[0]
                for t in range(1, DREG):
                    nv = nv + neg_s[r0 + k, pl.ds(t * LANES, LANES)] * cvecs[t]
                plsc.store_scatter(
                    negsc_v, [offv * KNEG + k], plsc.cumsum(nv), mask=m15)

        del bi_body

    issue(0, 0)

    def super_body(s, carry):
        c0 = s * 2
        issue(c0 + 1, 1)
        drain(0)
        compute(c0, 0)

        @pl.when(c0 + 2 < NCHUNK)
        def _():
            issue(c0 + 2, 0)

        drain(1)
        compute(c0 + 1, 1)
        return carry

    lax.fori_loop(0, NCHUNK // 2, super_body, 0)

    pltpu.sync_copy(pos_v, pos_out.at[pl.ds(base, BPW)])
    pltpu.sync_copy(negsc_v, neg_out.at[pl.ds(base * KNEG, BPW * KNEG)])


@functools.partial(
    pl.kernel,
    mesh=plsc.VectorSubcoreMesh(core_axis_name="c", subcore_axis_name="s"),
    out_type=(jax.ShapeDtypeStruct((BATCH,), jnp.float32),
              jax.ShapeDtypeStruct((BATCH * KNEG,), jnp.float32)),
    scratch_types=[
        pltpu.VMEM((BPW * KNEG,), jnp.int32),
        pltpu.VMEM((CHUNK, DIM), jnp.float32),
        pltpu.VMEM((CHUNK, DIM), jnp.float32),
        pltpu.VMEM((NEGC, DIM), jnp.float32),
        pltpu.VMEM((CHUNK, DIM), jnp.float32),
        pltpu.VMEM((CHUNK, DIM), jnp.float32),
        pltpu.VMEM((NEGC, DIM), jnp.float32),
        pltpu.VMEM((BPW,), jnp.float32),
        pltpu.VMEM((BPW * KNEG,), jnp.float32),
        pltpu.SemaphoreType.DMA,
        pltpu.SemaphoreType.DMA,
    ],
    compiler_params=pltpu.CompilerParams(
        use_tc_tiling_on_sc=True, needs_layout_passes=False),
)
def _skipgram_sc(cen_emb_hbm, ctx_emb_hbm, neg_hbm, wout_hbm,
                 pos_out, neg_out, *rest):
    _skipgram_body(cen_emb_hbm, ctx_emb_hbm, neg_hbm, wout_hbm,
                   pos_out, neg_out, *rest)


def kernel(center_words, context_words, negative_words, W_in, W_out):
    neg = negative_words.astype(jnp.int32).reshape(BATCH * KNEG)
    cen_emb = jnp.take(W_in, center_words, axis=0)
    ctx_emb = jnp.take(W_out, context_words, axis=0)
    pos, negsc = _skipgram_sc(cen_emb, ctx_emb, neg, W_out)
    return (pos, negsc.reshape(BATCH, KNEG))
